# Initial kernel scaffold; baseline (speedup 1.0000x reference)
import jax
import jax.numpy as jnp
from jax.experimental import pallas as pl


def kernel(boxes, scores, classes):
    raise NotImplementedError("write your pallas kernel here")


# speedup vs baseline: 16.4127x; 16.4127x over previous
import functools

import jax
import jax.numpy as jnp
from jax import lax
from jax.experimental import pallas as pl
from jax.experimental.pallas import tpu as pltpu
from jax.experimental.pallas import tpu_sc as plsc

_NUM_CLASSES = 80
_CONF = 0.25
_IOU_THR = 0.45
_MAX_DET = 300
_N_BOXES = 20000

_NT = 16
_L = 16
_CH = 1264
_NV = _CH // _L
_NPAD = _NT * _CH
_OUT_PAD = 320
_BIG = 2**30


def _nms_call(x1, y1, x2, y2, s, c):
    mesh = plsc.VectorSubcoreMesh(
        core_axis_name="c", subcore_axis_name="s", num_cores=1,
        num_subcores=_NT)

    def body(x1h, y1h, x2h, y2h, sh, ch, out_h,
             x1v, y1v, x2v, y2v, sv, areav, clsv, stag, allv, outv, shp):
        tid = lax.axis_index("s")
        base = tid * _CH
        lane = lax.broadcasted_iota(jnp.int32, (_L,), 0)

        pltpu.sync_copy(x1h.at[pl.ds(base, _CH)], x1v)
        pltpu.sync_copy(y1h.at[pl.ds(base, _CH)], y1v)
        pltpu.sync_copy(x2h.at[pl.ds(base, _CH)], x2v)
        pltpu.sync_copy(y2h.at[pl.ds(base, _CH)], y2v)
        pltpu.sync_copy(sh.at[pl.ds(base, _CH)], sv)
        pltpu.sync_copy(ch.at[pl.ds(base, _CH)], clsv)

        neg2 = jnp.full((_L,), -2.0, jnp.float32)
        zero_i = jnp.zeros((_L,), jnp.int32)

        def init_i(i, carry):
            bestv, bidxv = carry
            sl = pl.ds(i * _L, _L)
            x = sv[sl]
            x = jnp.where(x > _CONF, x, -1.0)
            sv[sl] = x
            areav[sl] = (x2v[sl] - x1v[sl]) * (y2v[sl] - y1v[sl])
            gidx = base + i * _L + lane
            better = x > bestv
            return jnp.where(better, x, bestv), jnp.where(better, gidx, bidxv)

        carry0 = lax.fori_loop(0, _NV, init_i, (neg2, zero_i))

        def step(t, carry):
            bestv, bidxv = carry
            m_t = jnp.max(bestv)
            gidx_t = jnp.min(jnp.where(bestv == m_t, bidxv, _BIG))
            lidx = jnp.broadcast_to(gidx_t - base, (_L,))
            wx1l = plsc.load_gather(x1v, [lidx])
            wy1l = plsc.load_gather(y1v, [lidx])
            wx2l = plsc.load_gather(x2v, [lidx])
            wy2l = plsc.load_gather(y2v, [lidx])
            wal = plsc.load_gather(areav, [lidx])
            wcl = plsc.load_gather(clsv, [lidx])
            row = jnp.full((_L,), m_t, jnp.float32)
            row = jnp.where(lane == 1, gidx_t.astype(jnp.float32), row)
            row = jnp.where(lane == 2, wx1l, row)
            row = jnp.where(lane == 3, wy1l, row)
            row = jnp.where(lane == 4, wx2l, row)
            row = jnp.where(lane == 5, wy2l, row)
            row = jnp.where(lane == 6, wal, row)
            row = jnp.where(lane == 7, wcl, row)
            stag[...] = row
            pltpu.sync_copy(stag, shp.at[pl.ds(tid * _L, _L)])
            plsc.subcore_barrier()
            pltpu.sync_copy(shp, allv)
            plsc.subcore_barrier()

            msv = plsc.load_gather(allv, [lane * _L])
            gm = jnp.max(msv)
            wt = jnp.min(jnp.where(msv == gm, lane, jnp.int32(_L - 1)))
            wrow = plsc.load_gather(allv, [jnp.broadcast_to(wt * _L, (_L,)) + lane])

            def pick(j):
                return jnp.sum(jnp.where(lane == j, wrow, 0.0))

            widx = pick(1).astype(jnp.int32)
            wx1 = pick(2)
            wy1 = pick(3)
            wx2 = pick(4)
            wy2 = pick(5)
            wa = pick(6)
            wcls = pick(7)
            valid = gm > 0.0
            vf = jnp.where(valid, 1.0, 0.0)

            @pl.when(tid == 0)
            def _():
                ov = jnp.full((_L,), vf, jnp.float32)
                ov = jnp.where(lane == 0, wx1 * vf, ov)
                ov = jnp.where(lane == 1, wy1 * vf, ov)
                ov = jnp.where(lane == 2, wx2 * vf, ov)
                ov = jnp.where(lane == 3, wy2 * vf, ov)
                ov = jnp.where(lane == 4, jnp.where(valid, wcls, -1.0), ov)
                ov = jnp.where(lane == 5, jnp.where(valid, gm, 0.0), ov)
                plsc.store_scatter(outv, [t * 8 + lane], ov, mask=lane < 7)

            def sup_i(i, cc):
                bv, biv = cc
                sl = pl.ds(i * _L, _L)
                ix1 = jnp.maximum(wx1, x1v[sl])
                iy1 = jnp.maximum(wy1, y1v[sl])
                ix2 = jnp.minimum(wx2, x2v[sl])
                iy2 = jnp.minimum(wy2, y2v[sl])
                inter = (jnp.maximum(ix2 - ix1, 0.0)
                         * jnp.maximum(iy2 - iy1, 0.0))
                iou = inter / (wa + areav[sl] - inter + 1e-9)
                gidx = base + i * _L + lane
                sup = (iou > _IOU_THR) | (gidx == widx)
                s_new = jnp.where(sup, -1.0, sv[sl])
                sv[sl] = s_new
                better = s_new > bv
                return (jnp.where(better, s_new, bv),
                        jnp.where(better, gidx, biv))

            return lax.fori_loop(0, _NV, sup_i, (neg2, zero_i))

        lax.fori_loop(0, _MAX_DET, step, carry0)

        @pl.when(tid == 0)
        def _():
            pltpu.sync_copy(outv, out_h)

    call = pl.kernel(
        body,
        out_type=jax.ShapeDtypeStruct((_OUT_PAD * 8,), jnp.float32),
        mesh=mesh,
        compiler_params=pltpu.CompilerParams(needs_layout_passes=False),
        scratch_types=[
            pltpu.VMEM((_CH,), jnp.float32),
            pltpu.VMEM((_CH,), jnp.float32),
            pltpu.VMEM((_CH,), jnp.float32),
            pltpu.VMEM((_CH,), jnp.float32),
            pltpu.VMEM((_CH,), jnp.float32),
            pltpu.VMEM((_CH,), jnp.float32),
            pltpu.VMEM((_CH,), jnp.float32),
            pltpu.VMEM((_L,), jnp.float32),
            pltpu.VMEM((_NT * _L,), jnp.float32),
            pltpu.VMEM((_OUT_PAD * 8,), jnp.float32),
            pltpu.VMEM_SHARED((_NT * _L,), jnp.float32),
        ],
    )
    return call(x1, y1, x2, y2, s, c)


def kernel(boxes, scores, classes):
    pad = _NPAD - _N_BOXES
    x1 = jnp.pad(boxes[:, 0], (0, pad))
    y1 = jnp.pad(boxes[:, 1], (0, pad))
    x2 = jnp.pad(boxes[:, 2], (0, pad))
    y2 = jnp.pad(boxes[:, 3], (0, pad))
    s = jnp.pad(scores, (0, pad), constant_values=-1.0)
    c = jnp.pad(classes.astype(jnp.float32), (0, pad))

    out = _nms_call(x1, y1, x2, y2, s, c)
    out = out.reshape(_OUT_PAD, 8)[:_MAX_DET]
    boxes_b = out[:, 0:4]
    labels_b = out[:, 4].astype(jnp.int32)
    scores_b = out[:, 5]
    sel_valid = out[:, 6] > 0.0
    return boxes_b, labels_b, scores_b, sel_valid
